# baseline (speedup 1.0000x reference)
import functools

import jax
import jax.numpy as jnp
from jax import lax
from jax.experimental import pallas as pl
from jax.experimental.pallas import tpu as pltpu
from jax.experimental.pallas import tpu_sc as plsc

H = 2048
W = 2048
NCH_IN = 10
NCH_OUT = 8
L = 16

SC_ROWS = 1280
TC_ROWS = H - SC_ROWS
TC_BR = 64

NC = 2
NS = 16
NW = NC * NS
ROWS_W = SC_ROWS // NW
CR = 8
CC = 256
CPIX = CR * CC
STRIPES = ROWS_W // CR
COLCH = W // CC
N_CHUNKS = STRIPES * COLCH
GROUPS = CPIX // L
NBUF = 2


def _sc_band(x):
    mesh = plsc.VectorSubcoreMesh(core_axis_name="c", subcore_axis_name="s")

    @functools.partial(
        pl.kernel,
        out_type=jax.ShapeDtypeStruct((1, NCH_OUT, H, W), jnp.float32),
        mesh=mesh,
        scratch_types=[
            pltpu.VMEM((NBUF, NCH_IN, CR, CC), jnp.float32),
            pltpu.VMEM((NBUF, NCH_OUT, CR, CC), jnp.float32),
            pltpu.SemaphoreType.DMA,
            pltpu.SemaphoreType.DMA,
            pltpu.SemaphoreType.DMA,
            pltpu.SemaphoreType.DMA,
        ],
    )
    def k(x_hbm, out_hbm, in_v, out_v, si0, si1, so0, so1):
        cid = lax.axis_index("c")
        sid = lax.axis_index("s")
        wid = sid * NC + cid
        row_base = wid * ROWS_W
        s_in = (si0, si1)
        s_out = (so0, so1)

        def chunk_rc(j):
            r0 = row_base + (j >> 3) * CR
            c0 = (j & 7) * CC
            return r0, c0

        def in_copy(j, b):
            r0, c0 = chunk_rc(j)
            return pltpu.make_async_copy(
                x_hbm.at[0, :, pl.ds(r0, CR), pl.ds(c0, CC)],
                in_v.at[b], s_in[b])

        def out_copy(j, b):
            r0, c0 = chunk_rc(j)
            return pltpu.make_async_copy(
                out_v.at[b],
                out_hbm.at[0, :, pl.ds(r0, CR), pl.ds(c0, CC)],
                s_out[b])

        def compute(b):
            @plsc.parallel_loop(0, GROUPS, unroll=4)
            def group_body(i):
                r = i >> 4
                sl = pl.ds((i & 15) * L, L)
                v = [in_v[b, ch, r, sl] for ch in range(NCH_OUT)]
                m = jnp.maximum(v[0], v[1])
                m2 = jnp.maximum(v[2], v[3])
                m3 = jnp.maximum(v[4], v[5])
                m4 = jnp.maximum(v[6], v[7])
                m = jnp.maximum(jnp.maximum(m, m2), jnp.maximum(m3, m4))
                dx = in_v[b, 8, r, sl]
                dy = in_v[b, 9, r, sl]
                s2 = dx * dx + dy * dy
                bits = lax.bitcast_convert_type(s2, jnp.int32)
                y = lax.bitcast_convert_type(
                    jnp.int32(0x5F3759DF) - (bits >> 1), jnp.float32)
                h = 0.5 * s2
                y = y * (1.5 - h * y * y)
                mag = s2 * y
                for ch in range(NCH_OUT):
                    out_v[b, ch, r, sl] = jnp.where(v[ch] == m, mag, 0.0)

        for b in range(NBUF):
            in_copy(b, b).start()

        def loop_body(t, carry):
            for b in range(NBUF):
                jj = t * NBUF + b
                in_copy(jj, b).wait()

                @pl.when(jj >= NBUF)
                def _():
                    out_copy(jj - NBUF, b).wait()

                compute(b)
                out_copy(jj, b).start()

                @pl.when(jj + NBUF < N_CHUNKS)
                def _():
                    in_copy(jj + NBUF, b).start()
            return carry

        lax.fori_loop(0, N_CHUNKS // NBUF, loop_body, 0)
        for b in range(NBUF):
            out_copy(N_CHUNKS - NBUF + b, b).wait()

    return k(x)


def _tc_body(x_ref, o_ref):
    xb = x_ref[0]
    cos = xb[:NCH_OUT]
    dx = xb[NCH_OUT]
    dy = xb[NCH_OUT + 1]
    mag = jnp.sqrt(dx * dx + dy * dy)
    m = jnp.max(cos, axis=0)
    o_ref[0] = jnp.where(cos == m[None], mag[None], 0.0)


def _tc_band(x):
    blk_off = SC_ROWS // TC_BR
    return pl.pallas_call(
        _tc_body,
        grid=(TC_ROWS // TC_BR,),
        in_specs=[pl.BlockSpec((1, NCH_IN, TC_BR, W),
                               lambda i: (0, 0, blk_off + i, 0))],
        out_specs=pl.BlockSpec((1, NCH_OUT, TC_BR, W),
                               lambda i: (0, 0, i, 0)),
        out_shape=jax.ShapeDtypeStruct((1, NCH_OUT, TC_ROWS, W), jnp.float32),
    )(x)


def kernel(x):
    sc_out = _sc_band(x)
    tc_out = _tc_band(x)
    return lax.dynamic_update_slice(sc_out, tc_out, (0, 0, SC_ROWS, 0))


# speedup vs baseline: 1.1851x; 1.1851x over previous
import functools

import jax
import jax.numpy as jnp
from jax import lax
from jax.experimental import pallas as pl
from jax.experimental.pallas import tpu as pltpu
from jax.experimental.pallas import tpu_sc as plsc

H = 2048
W = 2048
NCH_IN = 10
NCH_OUT = 8
L = 16

NC = 2
NS = 16
NW = NC * NS
ROWS_W = H // NW
CR = 8
CC = 256
CPIX = CR * CC
STRIPES = ROWS_W // CR
COLCH = W // CC
N_CHUNKS = STRIPES * COLCH
GROUPS = CPIX // L
NBI = 4
NBO = 2


def _sc_histogram(x):
    mesh = plsc.VectorSubcoreMesh(core_axis_name="c", subcore_axis_name="s")

    @functools.partial(
        pl.kernel,
        out_type=jax.ShapeDtypeStruct((1, NCH_OUT, H, W), jnp.float32),
        mesh=mesh,
        scratch_types=[
            pltpu.VMEM((NBI, NCH_IN, CR, CC), jnp.float32),
            pltpu.VMEM((NBO, NCH_OUT, CR, CC), jnp.float32),
            pltpu.SemaphoreType.DMA,
            pltpu.SemaphoreType.DMA,
            pltpu.SemaphoreType.DMA,
            pltpu.SemaphoreType.DMA,
            pltpu.SemaphoreType.DMA,
            pltpu.SemaphoreType.DMA,
        ],
    )
    def k(x_hbm, out_hbm, in_v, out_v, si0, si1, si2, si3, so0, so1):
        cid = lax.axis_index("c")
        sid = lax.axis_index("s")
        wid = sid * NC + cid
        row_base = wid * ROWS_W
        s_in = (si0, si1, si2, si3)
        s_out = (so0, so1)

        def chunk_rc(j):
            r0 = row_base + (j >> 3) * CR
            c0 = (j & 7) * CC
            return r0, c0

        def in_copy(j, b):
            r0, c0 = chunk_rc(j)
            return pltpu.make_async_copy(
                x_hbm.at[0, :, pl.ds(r0, CR), pl.ds(c0, CC)],
                in_v.at[b], s_in[b])

        def out_copy(j, b):
            r0, c0 = chunk_rc(j)
            return pltpu.make_async_copy(
                out_v.at[b],
                out_hbm.at[0, :, pl.ds(r0, CR), pl.ds(c0, CC)],
                s_out[b])

        def compute(b, bo):
            @plsc.parallel_loop(0, GROUPS, unroll=4)
            def group_body(i):
                r = i >> 4
                sl = pl.ds((i & 15) * L, L)
                v = [in_v[b, ch, r, sl] for ch in range(NCH_OUT)]
                m = jnp.maximum(v[0], v[1])
                m2 = jnp.maximum(v[2], v[3])
                m3 = jnp.maximum(v[4], v[5])
                m4 = jnp.maximum(v[6], v[7])
                m = jnp.maximum(jnp.maximum(m, m2), jnp.maximum(m3, m4))
                dx = in_v[b, 8, r, sl]
                dy = in_v[b, 9, r, sl]
                s2 = dx * dx + dy * dy
                bits = lax.bitcast_convert_type(s2, jnp.int32)
                y = lax.bitcast_convert_type(
                    jnp.int32(0x5F3759DF) - (bits >> 1), jnp.float32)
                h = 0.5 * s2
                y = y * (1.5 - h * y * y)
                mag = s2 * y
                for ch in range(NCH_OUT):
                    out_v[bo, ch, r, sl] = jnp.where(v[ch] == m, mag, 0.0)

        for b in range(NBI):
            in_copy(b, b).start()

        def loop_body(t, carry):
            for b in range(NBI):
                jj = t * NBI + b
                bo = b % NBO
                in_copy(jj, b).wait()

                @pl.when(jj >= NBO)
                def _():
                    out_copy(jj - NBO, bo).wait()

                compute(b, bo)
                out_copy(jj, bo).start()

                @pl.when(jj + NBI < N_CHUNKS)
                def _():
                    in_copy(jj + NBI, b).start()
            return carry

        lax.fori_loop(0, N_CHUNKS // NBI, loop_body, 0)
        for j in range(N_CHUNKS - NBO, N_CHUNKS):
            out_copy(j, j % NBO).wait()

    return k(x)


def kernel(x):
    return _sc_histogram(x)
